# baseline (speedup 1.0000x reference)
import functools

import jax
import jax.numpy as jnp
from jax import lax
from jax.experimental import pallas as pl
from jax.experimental.pallas import tpu as pltpu
from jax.experimental.pallas import tpu_sc as plsc

B, T, V = 16, 256, 96
W = 16
BLANK = V - 1
NV = V // 16


VP = 128


def _ls_body(x_ref, o_ref):
    x = x_ref[...]
    m = jnp.max(x, axis=-1, keepdims=True)
    s = x - m
    lp = s - jnp.log(jnp.sum(jnp.exp(s), axis=-1, keepdims=True))
    o_ref[...] = jnp.pad(lp, ((0, 0), (0, VP - V)))


_LS_GRID = 2


def _log_softmax_tc(x):
    rows = B * T // _LS_GRID
    return pl.pallas_call(
        _ls_body,
        grid=(_LS_GRID,),
        in_specs=[pl.BlockSpec((rows, V), lambda i: (i, 0))],
        out_specs=pl.BlockSpec((rows, VP), lambda i: (i, 0)),
        out_shape=jax.ShapeDtypeStruct((B * T, VP), jnp.float32),
    )(x)


def _take16(v, idx):
    return jnp.take_along_axis(v, idx, axis=0, mode="promise_in_bounds")


def _sortd(a):
    return plsc.sort_key_val(a[0], a[1], descending=True)


def _sorta(a):
    return plsc.sort_key_val(a[0], a[1], descending=False)


def _merge_ad(a, b):
    av, ap = a
    bv, bp = b
    cm = av >= bv
    return jnp.where(cm, av, bv), jnp.where(cm, ap, bp)


def _sc_body(logp_hbm, dec_hbm, prob_hbm, logp_v, path_v, rec_v, dec_v, prob_v):
    cid = lax.axis_index("c")
    sid = lax.axis_index("s")

    @pl.when(cid == 0)
    def _():
        b = sid
        pltpu.sync_copy(logp_hbm.at[pl.ds(b * T * VP, T * VP)], logp_v)
        iota = lax.iota(jnp.int32, 16)
        riota = 15 - iota

        def top16x(t):
            parts = []
            for i in range(NV):
                v = logp_v[pl.ds(t * VP + 16 * i, 16)]
                p = iota + 16 * i
                parts.append((v, p))
            s0, s1 = _sortd(parts[0]), _sorta(parts[1])
            s2, s3 = _sortd(parts[2]), _sorta(parts[3])
            s4, s5 = _sortd(parts[4]), _sorta(parts[5])
            m01 = _sortd(_merge_ad(s0, s1))
            m23 = _sorta(_merge_ad(s2, s3))
            m45 = _sorta(_merge_ad(s4, s5))
            m0123 = _sortd(_merge_ad(m01, m23))
            fin = _merge_ad(m0123, m45)
            lptv, lpti = _sortd(fin)
            lptva, _ = _sorta(fin)
            return lptv, lptva, lpti

        def beam_update(t, scores, abc):
            lptv, lptva, lpti = abc
            lvl = []
            for k in range(8):
                we, wo = 2 * k, 2 * k + 1
                se = _take16(scores, jnp.full((16,), we, jnp.int32))
                so = _take16(scores, jnp.full((16,), wo, jnp.int32))
                m = _merge_ad((se + lptv, iota + 16 * we),
                              (so + lptva, riota + 16 * wo))
                lvl.append(_sortd(m) if k % 2 == 0 else _sorta(m))
            lvl = [_merge_ad(lvl[2 * k], lvl[2 * k + 1]) for k in range(4)]
            lvl = [_sortd(m) if k % 2 == 0 else _sorta(m)
                   for k, m in enumerate(lvl)]
            lvl = [_merge_ad(lvl[0], lvl[1]), _merge_ad(lvl[2], lvl[3])]
            rv, rp = _merge_ad(_sortd(lvl[0]), _sorta(lvl[1]))
            wpar = rp >> 4
            j = rp & 15
            tok = _take16(lpti, j)
            rec_v[pl.ds(t * 16, 16)] = (wpar << 7) | tok
            return rv

        lptv0, _, lpti0 = top16x(0)
        rec_v[pl.ds(0, 16)] = lpti0

        def step(t, carry):
            scores, abc = carry
            nxt = top16x(t + 1)
            rv = beam_update(t, scores, abc)
            return rv, nxt

        scores, abc = lax.fori_loop(1, T - 1, step, (lptv0, top16x(1)),
                                    unroll=2)
        scores = beam_update(T - 1, scores, abc)
        scores, border = _sortd((scores, iota))

        lane0 = iota == 0

        def bstep(k, wv):
            t = T - 1 - k
            r = plsc.load_gather(rec_v, [jnp.full((16,), t * 16, jnp.int32) + wv])
            plsc.store_scatter(path_v, [jnp.full((16,), t, jnp.int32)],
                               r & 127, mask=lane0)
            return r >> 7

        wv = lax.fori_loop(0, T - 1, bstep,
                           _take16(border, jnp.zeros((16,), jnp.int32)),
                           unroll=4)
        r0 = plsc.load_gather(rec_v, [wv])
        plsc.store_scatter(path_v, [jnp.zeros((16,), jnp.int32)],
                           r0 & 127, mask=lane0)

        for i in range(T // 16):
            dec_v[pl.ds(16 * i, 16)] = jnp.full((16,), -1, jnp.int32)
        running = jnp.int32(0)
        for i in range(T // 16):
            cur = path_v[pl.ds(16 * i, 16)]
            if i == 0:
                prev = plsc.load_gather(path_v, [jnp.maximum(iota - 1, 0)])
                prev = jnp.where(lane0, -1, prev)
            else:
                prev = plsc.load_gather(path_v, [iota + (16 * i - 1)])
            keep = (cur != prev) & (cur != BLANK)
            kint = jnp.where(keep, 1, 0).astype(jnp.int32)
            pos = plsc.cumsum(kint) + running - 1
            plsc.store_scatter(dec_v, [pos], cur, mask=keep)
            running = running + jnp.sum(kint)

        prob_v[...] = jnp.exp(scores)
        pltpu.sync_copy(dec_v, dec_hbm.at[pl.ds(b * T, T)])
        pltpu.sync_copy(prob_v, prob_hbm.at[pl.ds(b * 16, 16)])


@functools.cache
def _sc_decode():
    return pl.kernel(
        _sc_body,
        out_type=[
            jax.ShapeDtypeStruct((B * T,), jnp.int32),
            jax.ShapeDtypeStruct((B * 16,), jnp.float32),
        ],
        mesh=plsc.VectorSubcoreMesh(core_axis_name="c", subcore_axis_name="s"),
        compiler_params=pltpu.CompilerParams(needs_layout_passes=False),
        scratch_types=[
            pltpu.VMEM((T * VP,), jnp.float32),
            pltpu.VMEM((T,), jnp.int32),
            pltpu.VMEM((T * 16,), jnp.int32),
            pltpu.VMEM((T,), jnp.int32),
            pltpu.VMEM((16,), jnp.float32),
        ],
    )


def kernel(inputs):
    logp = _log_softmax_tc(inputs.reshape(B * T, V))
    dec, prob = _sc_decode()(logp.reshape(B * T * VP))
    decoded = dec.reshape(B, 1, T)
    probability = prob.reshape(B, 16)[:, :1]
    return decoded, probability


# speedup vs baseline: 1.1559x; 1.1559x over previous
import functools

import jax
import jax.numpy as jnp
from jax import lax
from jax.experimental import pallas as pl
from jax.experimental.pallas import tpu as pltpu
from jax.experimental.pallas import tpu_sc as plsc

B, T, V = 16, 256, 96
W = 16
BLANK = V - 1
NV = V // 16


VP = 128


def _ls_body(x_ref, o_ref):
    x = x_ref[...]
    m = jnp.max(x, axis=-1, keepdims=True)
    s = x - m
    lp = s - jnp.log(jnp.sum(jnp.exp(s), axis=-1, keepdims=True))
    o_ref[...] = jnp.pad(lp, ((0, 0), (0, VP - V)))


_LS_GRID = 2


def _log_softmax_tc(x):
    rows = B * T // _LS_GRID
    return pl.pallas_call(
        _ls_body,
        grid=(_LS_GRID,),
        in_specs=[pl.BlockSpec((rows, V), lambda i: (i, 0))],
        out_specs=pl.BlockSpec((rows, VP), lambda i: (i, 0)),
        out_shape=jax.ShapeDtypeStruct((B * T, VP), jnp.float32),
    )(x)


def _take16(v, idx):
    return jnp.take_along_axis(v, idx, axis=0, mode="promise_in_bounds")


def _sortd(a):
    return plsc.sort_key_val(a[0], a[1], descending=True)


def _sorta(a):
    return plsc.sort_key_val(a[0], a[1], descending=False)


def _merge_ad(a, b):
    av, ap = a
    bv, bp = b
    cm = av >= bv
    return jnp.where(cm, av, bv), jnp.where(cm, ap, bp)


def _sc_body(logp_hbm, dec_hbm, prob_hbm, logp_v, path_v, rec_v, dec_v, prob_v):
    cid = lax.axis_index("c")
    sid = lax.axis_index("s")

    @pl.when(cid == 0)
    def _():
        b = sid
        pltpu.sync_copy(logp_hbm.at[pl.ds(b * T * VP, T * VP)], logp_v)
        iota = lax.iota(jnp.int32, 16)
        riota = 15 - iota

        def top16x(t):
            parts = []
            for i in range(NV):
                v = logp_v[pl.ds(t * VP + 16 * i, 16)]
                p = iota + 16 * i
                parts.append((v, p))
            s0, s1 = _sortd(parts[0]), _sorta(parts[1])
            s2, s3 = _sortd(parts[2]), _sorta(parts[3])
            s4, s5 = _sortd(parts[4]), _sorta(parts[5])
            m01 = _sortd(_merge_ad(s0, s1))
            m23 = _sorta(_merge_ad(s2, s3))
            m45 = _sorta(_merge_ad(s4, s5))
            m0123 = _sortd(_merge_ad(m01, m23))
            fin = _merge_ad(m0123, m45)
            lptv, lpti = _sortd(fin)
            lptva, _ = _sorta(fin)
            return lptv, lptva, lpti

        def beam_update(t, scores, abc):
            lptv, lptva, lpti = abc
            lvl = []
            for k in range(8):
                we, wo = 2 * k, 2 * k + 1
                se = _take16(scores, jnp.full((16,), we, jnp.int32))
                so = _take16(scores, jnp.full((16,), wo, jnp.int32))
                m = _merge_ad((se + lptv, iota + 16 * we),
                              (so + lptva, riota + 16 * wo))
                lvl.append(_sortd(m) if k % 2 == 0 else _sorta(m))
            lvl = [_merge_ad(lvl[2 * k], lvl[2 * k + 1]) for k in range(4)]
            lvl = [_sortd(m) if k % 2 == 0 else _sorta(m)
                   for k, m in enumerate(lvl)]
            lvl = [_merge_ad(lvl[0], lvl[1]), _merge_ad(lvl[2], lvl[3])]
            rv, rp = _merge_ad(_sortd(lvl[0]), _sorta(lvl[1]))
            wpar = rp >> 4
            j = rp & 15
            tok = _take16(lpti, j)
            rec_v[pl.ds(t * 16, 16)] = (wpar << 7) | tok
            return rv

        lptv0, _, lpti0 = top16x(0)
        rec_v[pl.ds(0, 16)] = lpti0

        def step(t, carry):
            scores, abc = carry
            nxt = top16x(t + 1)
            rv = beam_update(t, scores, abc)
            return rv, nxt

        scores, abc = lax.fori_loop(1, T - 1, step, (lptv0, top16x(1)))
        scores = beam_update(T - 1, scores, abc)
        scores, border = _sortd((scores, iota))

        lane0 = iota == 0

        def bstep(k, wv):
            t = T - 1 - k
            r = plsc.load_gather(rec_v, [jnp.full((16,), t * 16, jnp.int32) + wv])
            plsc.store_scatter(path_v, [jnp.full((16,), t, jnp.int32)],
                               r & 127, mask=lane0)
            return r >> 7

        wv = lax.fori_loop(0, T - 1, bstep,
                           _take16(border, jnp.zeros((16,), jnp.int32)))
        r0 = plsc.load_gather(rec_v, [wv])
        plsc.store_scatter(path_v, [jnp.zeros((16,), jnp.int32)],
                           r0 & 127, mask=lane0)

        for i in range(T // 16):
            dec_v[pl.ds(16 * i, 16)] = jnp.full((16,), -1, jnp.int32)
        running = jnp.int32(0)
        for i in range(T // 16):
            cur = path_v[pl.ds(16 * i, 16)]
            if i == 0:
                prev = plsc.load_gather(path_v, [jnp.maximum(iota - 1, 0)])
                prev = jnp.where(lane0, -1, prev)
            else:
                prev = plsc.load_gather(path_v, [iota + (16 * i - 1)])
            keep = (cur != prev) & (cur != BLANK)
            kint = jnp.where(keep, 1, 0).astype(jnp.int32)
            pos = plsc.cumsum(kint) + running - 1
            plsc.store_scatter(dec_v, [pos], cur, mask=keep)
            running = running + jnp.sum(kint)

        prob_v[...] = jnp.exp(scores)
        pltpu.sync_copy(dec_v, dec_hbm.at[pl.ds(b * T, T)])
        pltpu.sync_copy(prob_v, prob_hbm.at[pl.ds(b * 16, 16)])


@functools.cache
def _sc_decode():
    return pl.kernel(
        _sc_body,
        out_type=[
            jax.ShapeDtypeStruct((B * T,), jnp.int32),
            jax.ShapeDtypeStruct((B * 16,), jnp.float32),
        ],
        mesh=plsc.VectorSubcoreMesh(core_axis_name="c", subcore_axis_name="s"),
        compiler_params=pltpu.CompilerParams(needs_layout_passes=False),
        scratch_types=[
            pltpu.VMEM((T * VP,), jnp.float32),
            pltpu.VMEM((T,), jnp.int32),
            pltpu.VMEM((T * 16,), jnp.int32),
            pltpu.VMEM((T,), jnp.int32),
            pltpu.VMEM((16,), jnp.float32),
        ],
    )


def kernel(inputs):
    logp = _log_softmax_tc(inputs.reshape(B * T, V))
    dec, prob = _sc_decode()(logp.reshape(B * T * VP))
    decoded = dec.reshape(B, 1, T)
    probability = prob.reshape(B, 16)[:, :1]
    return decoded, probability


# speedup vs baseline: 1.1591x; 1.0028x over previous
import functools

import jax
import jax.numpy as jnp
from jax import lax
from jax.experimental import pallas as pl
from jax.experimental.pallas import tpu as pltpu
from jax.experimental.pallas import tpu_sc as plsc

B, T, V = 16, 256, 96
W = 16
BLANK = V - 1
NV = V // 16


VP = 128


def _ls_body(x_ref, o_ref):
    x = x_ref[...]
    m = jnp.max(x, axis=-1, keepdims=True)
    s = x - m
    lp = s - jnp.log(jnp.sum(jnp.exp(s), axis=-1, keepdims=True))
    o_ref[...] = jnp.pad(lp, ((0, 0), (0, VP - V)))


_LS_GRID = 2


def _log_softmax_tc(x):
    rows = B * T // _LS_GRID
    return pl.pallas_call(
        _ls_body,
        grid=(_LS_GRID,),
        in_specs=[pl.BlockSpec((rows, V), lambda i: (i, 0))],
        out_specs=pl.BlockSpec((rows, VP), lambda i: (i, 0)),
        out_shape=jax.ShapeDtypeStruct((B * T, VP), jnp.float32),
    )(x)


def _take16(v, idx):
    return jnp.take_along_axis(v, idx, axis=0, mode="promise_in_bounds")


def _sortd(a):
    return plsc.sort_key_val(a[0], a[1], descending=True)


def _sorta(a):
    return plsc.sort_key_val(a[0], a[1], descending=False)


def _merge_ad(a, b):
    av, ap = a
    bv, bp = b
    cm = av >= bv
    return jnp.where(cm, av, bv), jnp.where(cm, ap, bp)


def _sc_body(logp_hbm, dec_hbm, prob_hbm, logp_v, path_v, rec_v, dec_v, prob_v):
    cid = lax.axis_index("c")
    sid = lax.axis_index("s")

    @pl.when(cid == 0)
    def _():
        b = sid
        pltpu.sync_copy(logp_hbm.at[pl.ds(b * T * VP, T * VP)], logp_v)
        iota = lax.iota(jnp.int32, 16)
        riota = 15 - iota

        def top16x(t):
            parts = []
            for i in range(NV):
                v = logp_v[pl.ds(t * VP + 16 * i, 16)]
                p = iota + 16 * i
                parts.append((v, p))
            s0, s1 = _sortd(parts[0]), _sorta(parts[1])
            s2, s3 = _sortd(parts[2]), _sorta(parts[3])
            s4, s5 = _sortd(parts[4]), _sorta(parts[5])
            m01 = _sortd(_merge_ad(s0, s1))
            m23 = _sorta(_merge_ad(s2, s3))
            m45 = _sorta(_merge_ad(s4, s5))
            m0123 = _sortd(_merge_ad(m01, m23))
            fin = _merge_ad(m0123, m45)
            lptv, lpti = _sortd(fin)
            lptva, _ = _sorta(fin)
            return lptv, lptva, lpti

        def beam_update(t, scores, abc):
            lptv, lptva, lpti = abc
            lvl = []
            for k in range(8):
                we, wo = 2 * k, 2 * k + 1
                se = _take16(scores, jnp.full((16,), we, jnp.int32))
                so = _take16(scores, jnp.full((16,), wo, jnp.int32))
                m = _merge_ad((se + lptv, iota + 16 * we),
                              (so + lptva, riota + 16 * wo))
                lvl.append(_sortd(m) if k % 2 == 0 else _sorta(m))
            lvl = [_merge_ad(lvl[2 * k], lvl[2 * k + 1]) for k in range(4)]
            lvl = [_sortd(m) if k % 2 == 0 else _sorta(m)
                   for k, m in enumerate(lvl)]
            lvl = [_merge_ad(lvl[0], lvl[1]), _merge_ad(lvl[2], lvl[3])]
            rv, rp = _merge_ad(_sortd(lvl[0]), _sorta(lvl[1]))
            wpar = rp >> 4
            j = rp & 15
            tok = _take16(lpti, j)
            rec_v[pl.ds(t * 16, 16)] = (wpar << 7) | tok
            return rv

        lptv0, _, lpti0 = top16x(0)
        rec_v[pl.ds(0, 16)] = lpti0

        def step(t, carry):
            scores, abc = carry
            nxt = top16x(jnp.minimum(t + 1, T - 1))
            rv = beam_update(t, scores, abc)
            return rv, nxt

        scores, abc = lax.fori_loop(1, T, step, (lptv0, top16x(1)))
        scores, border = _sortd((scores, iota))

        lane0 = iota == 0

        def bstep(k, wv):
            t = T - 1 - k
            r = plsc.load_gather(rec_v, [jnp.full((16,), t * 16, jnp.int32) + wv])
            plsc.store_scatter(path_v, [jnp.full((16,), t, jnp.int32)],
                               r & 127, mask=lane0)
            return r >> 7

        wv = lax.fori_loop(0, T - 1, bstep,
                           _take16(border, jnp.zeros((16,), jnp.int32)))
        r0 = plsc.load_gather(rec_v, [wv])
        plsc.store_scatter(path_v, [jnp.zeros((16,), jnp.int32)],
                           r0 & 127, mask=lane0)

        for i in range(T // 16):
            dec_v[pl.ds(16 * i, 16)] = jnp.full((16,), -1, jnp.int32)
        running = jnp.int32(0)
        for i in range(T // 16):
            cur = path_v[pl.ds(16 * i, 16)]
            if i == 0:
                prev = plsc.load_gather(path_v, [jnp.maximum(iota - 1, 0)])
                prev = jnp.where(lane0, -1, prev)
            else:
                prev = plsc.load_gather(path_v, [iota + (16 * i - 1)])
            keep = (cur != prev) & (cur != BLANK)
            kint = jnp.where(keep, 1, 0).astype(jnp.int32)
            pos = plsc.cumsum(kint) + running - 1
            plsc.store_scatter(dec_v, [pos], cur, mask=keep)
            running = running + jnp.sum(kint)

        prob_v[...] = jnp.exp(scores)
        pltpu.sync_copy(dec_v, dec_hbm.at[pl.ds(b * T, T)])
        pltpu.sync_copy(prob_v, prob_hbm.at[pl.ds(b * 16, 16)])


@functools.cache
def _sc_decode():
    return pl.kernel(
        _sc_body,
        out_type=[
            jax.ShapeDtypeStruct((B * T,), jnp.int32),
            jax.ShapeDtypeStruct((B * 16,), jnp.float32),
        ],
        mesh=plsc.VectorSubcoreMesh(core_axis_name="c", subcore_axis_name="s"),
        compiler_params=pltpu.CompilerParams(needs_layout_passes=False),
        scratch_types=[
            pltpu.VMEM((T * VP,), jnp.float32),
            pltpu.VMEM((T,), jnp.int32),
            pltpu.VMEM((T * 16,), jnp.int32),
            pltpu.VMEM((T,), jnp.int32),
            pltpu.VMEM((16,), jnp.float32),
        ],
    )


def kernel(inputs):
    logp = _log_softmax_tc(inputs.reshape(B * T, V))
    dec, prob = _sc_decode()(logp.reshape(B * T * VP))
    decoded = dec.reshape(B, 1, T)
    probability = prob.reshape(B, 16)[:, :1]
    return decoded, probability


# speedup vs baseline: 1.1603x; 1.0011x over previous
import functools

import jax
import jax.numpy as jnp
from jax import lax
from jax.experimental import pallas as pl
from jax.experimental.pallas import tpu as pltpu
from jax.experimental.pallas import tpu_sc as plsc

B, T, V = 16, 256, 96
W = 16
BLANK = V - 1
NV = V // 16


VP = 128


def _ls_body(x_ref, o_ref):
    x = x_ref[...]
    m = jnp.max(x, axis=-1, keepdims=True)
    s = x - m
    lp = s - jnp.log(jnp.sum(jnp.exp(s), axis=-1, keepdims=True))
    o_ref[...] = jnp.pad(lp, ((0, 0), (0, VP - V)))


_LS_GRID = 2


def _log_softmax_tc(x):
    rows = B * T // _LS_GRID
    return pl.pallas_call(
        _ls_body,
        grid=(_LS_GRID,),
        in_specs=[pl.BlockSpec((rows, V), lambda i: (i, 0))],
        out_specs=pl.BlockSpec((rows, VP), lambda i: (i, 0)),
        out_shape=jax.ShapeDtypeStruct((B * T, VP), jnp.float32),
    )(x)


def _take16(v, idx):
    return jnp.take_along_axis(v, idx, axis=0, mode="promise_in_bounds")


def _sortd(a):
    return plsc.sort_key_val(a[0], a[1], descending=True)


def _sorta(a):
    return plsc.sort_key_val(a[0], a[1], descending=False)


def _merge_ad(a, b):
    av, ap = a
    bv, bp = b
    cm = av >= bv
    return jnp.where(cm, av, bv), jnp.where(cm, ap, bp)


def _sc_body(logp_hbm, dec_hbm, prob_hbm, logp_v, path_v, rec_v, dec_v, prob_v):
    cid = lax.axis_index("c")
    sid = lax.axis_index("s")

    @pl.when(sid < 8)
    def _():
        b = cid * 8 + sid
        pltpu.sync_copy(logp_hbm.at[pl.ds(b * T * VP, T * VP)], logp_v)
        iota = lax.iota(jnp.int32, 16)
        riota = 15 - iota

        def top16x(t):
            parts = []
            for i in range(NV):
                v = logp_v[pl.ds(t * VP + 16 * i, 16)]
                p = iota + 16 * i
                parts.append((v, p))
            s0, s1 = _sortd(parts[0]), _sorta(parts[1])
            s2, s3 = _sortd(parts[2]), _sorta(parts[3])
            s4, s5 = _sortd(parts[4]), _sorta(parts[5])
            m01 = _sortd(_merge_ad(s0, s1))
            m23 = _sorta(_merge_ad(s2, s3))
            m45 = _sorta(_merge_ad(s4, s5))
            m0123 = _sortd(_merge_ad(m01, m23))
            fin = _merge_ad(m0123, m45)
            lptv, lpti = _sortd(fin)
            lptva, _ = _sorta(fin)
            return lptv, lptva, lpti

        def beam_update(t, scores, abc):
            lptv, lptva, lpti = abc
            lvl = []
            for k in range(8):
                we, wo = 2 * k, 2 * k + 1
                se = _take16(scores, jnp.full((16,), we, jnp.int32))
                so = _take16(scores, jnp.full((16,), wo, jnp.int32))
                m = _merge_ad((se + lptv, iota + 16 * we),
                              (so + lptva, riota + 16 * wo))
                lvl.append(_sortd(m) if k % 2 == 0 else _sorta(m))
            lvl = [_merge_ad(lvl[2 * k], lvl[2 * k + 1]) for k in range(4)]
            lvl = [_sortd(m) if k % 2 == 0 else _sorta(m)
                   for k, m in enumerate(lvl)]
            lvl = [_merge_ad(lvl[0], lvl[1]), _merge_ad(lvl[2], lvl[3])]
            rv, rp = _merge_ad(_sortd(lvl[0]), _sorta(lvl[1]))
            wpar = rp >> 4
            j = rp & 15
            tok = _take16(lpti, j)
            rec_v[pl.ds(t * 16, 16)] = (wpar << 7) | tok
            return rv

        lptv0, _, lpti0 = top16x(0)
        rec_v[pl.ds(0, 16)] = lpti0

        def step(t, carry):
            scores, abc = carry
            nxt = top16x(jnp.minimum(t + 1, T - 1))
            rv = beam_update(t, scores, abc)
            return rv, nxt

        scores, abc = lax.fori_loop(1, T, step, (lptv0, top16x(1)))
        scores, border = _sortd((scores, iota))

        lane0 = iota == 0

        def bstep(k, wv):
            t = T - 1 - k
            r = plsc.load_gather(rec_v, [jnp.full((16,), t * 16, jnp.int32) + wv])
            plsc.store_scatter(path_v, [jnp.full((16,), t, jnp.int32)],
                               r & 127, mask=lane0)
            return r >> 7

        wv = lax.fori_loop(0, T - 1, bstep,
                           _take16(border, jnp.zeros((16,), jnp.int32)))
        r0 = plsc.load_gather(rec_v, [wv])
        plsc.store_scatter(path_v, [jnp.zeros((16,), jnp.int32)],
                           r0 & 127, mask=lane0)

        for i in range(T // 16):
            dec_v[pl.ds(16 * i, 16)] = jnp.full((16,), -1, jnp.int32)
        running = jnp.int32(0)
        for i in range(T // 16):
            cur = path_v[pl.ds(16 * i, 16)]
            if i == 0:
                prev = plsc.load_gather(path_v, [jnp.maximum(iota - 1, 0)])
                prev = jnp.where(lane0, -1, prev)
            else:
                prev = plsc.load_gather(path_v, [iota + (16 * i - 1)])
            keep = (cur != prev) & (cur != BLANK)
            kint = jnp.where(keep, 1, 0).astype(jnp.int32)
            pos = plsc.cumsum(kint) + running - 1
            plsc.store_scatter(dec_v, [pos], cur, mask=keep)
            running = running + jnp.sum(kint)

        prob_v[...] = jnp.exp(scores)
        pltpu.sync_copy(dec_v, dec_hbm.at[pl.ds(b * T, T)])
        pltpu.sync_copy(prob_v, prob_hbm.at[pl.ds(b * 16, 16)])


@functools.cache
def _sc_decode():
    return pl.kernel(
        _sc_body,
        out_type=[
            jax.ShapeDtypeStruct((B * T,), jnp.int32),
            jax.ShapeDtypeStruct((B * 16,), jnp.float32),
        ],
        mesh=plsc.VectorSubcoreMesh(core_axis_name="c", subcore_axis_name="s"),
        compiler_params=pltpu.CompilerParams(needs_layout_passes=False),
        scratch_types=[
            pltpu.VMEM((T * VP,), jnp.float32),
            pltpu.VMEM((T,), jnp.int32),
            pltpu.VMEM((T * 16,), jnp.int32),
            pltpu.VMEM((T,), jnp.int32),
            pltpu.VMEM((16,), jnp.float32),
        ],
    )


def kernel(inputs):
    logp = _log_softmax_tc(inputs.reshape(B * T, V))
    dec, prob = _sc_decode()(logp.reshape(B * T * VP))
    decoded = dec.reshape(B, 1, T)
    probability = prob.reshape(B, 16)[:, :1]
    return decoded, probability
